# baseline (device time: 62013 ns/iter reference)
import jax
import jax.numpy as jnp
from jax import lax
from jax.experimental import pallas as pl
from jax.experimental.pallas import tpu as pltpu

M_PER = 4096
N = 1024
HALF = M_PER // 2

CHUNKS = [32] * 4 + [128] * 14 + [32] * 4
OFFS = [sum(CHUNKS[:i]) for i in range(len(CHUNKS))]
NC = len(CHUNKS)
assert OFFS[-1] + CHUNKS[-1] == HALF

SUB = 4
SUBR = HALF // SUB


def kernel(x):
    def body(x_ref, out_ref, xvm, buf, rbuf, send_a, recv_a, send_b, recv_b,
             in_sems, local_sem, rcopy_sems):
        my_x = lax.axis_index("x")
        my_y = lax.axis_index("y")
        other_x = 1 - my_x
        other_y = 1 - my_y

        in_dmas = []
        for j in range(2 * SUB):
            half = my_y * HALF if j < SUB else other_y * HALF
            r0 = half + (j % SUB) * SUBR
            c = pltpu.make_async_copy(
                x_ref.at[pl.ds(r0, SUBR), :], xvm.at[pl.ds(r0, SUBR), :],
                in_sems.at[j],
            )
            c.start()
            in_dmas.append(c)

        barrier_sem = pltpu.get_barrier_semaphore()
        pl.semaphore_signal(barrier_sem, inc=1, device_id=(other_x, my_y),
                            device_id_type=pl.DeviceIdType.MESH)
        pl.semaphore_signal(barrier_sem, inc=1, device_id=(my_x, other_y),
                            device_id_type=pl.DeviceIdType.MESH)
        pl.semaphore_wait(barrier_sem, 2)

        my_half_rows = my_x * M_PER + my_y * HALF
        rdma_a = []
        subs_waited = 0
        for i in range(NC):
            need = (OFFS[i] + CHUNKS[i] + SUBR - 1) // SUBR
            while subs_waited < need:
                in_dmas[subs_waited].wait()
                subs_waited += 1
            r0 = my_y * HALF + OFFS[i]
            buf[pl.ds(r0, CHUNKS[i]), :] = (
                xvm[pl.ds(r0, CHUNKS[i]), :].astype(jnp.bfloat16)
            )
            r = pltpu.make_async_remote_copy(
                src_ref=buf.at[pl.ds(r0, CHUNKS[i]), :],
                dst_ref=rbuf.at[pl.ds(OFFS[i], CHUNKS[i]), :],
                send_sem=send_a.at[i],
                recv_sem=recv_a.at[i],
                device_id=(other_x, my_y),
                device_id_type=pl.DeviceIdType.MESH,
            )
            r.start()
            rdma_a.append(r)

        for j in range(SUB):
            in_dmas[SUB + j].wait()
            r0 = other_y * HALF + j * SUBR
            buf[pl.ds(r0, SUBR), :] = (
                xvm[pl.ds(r0, SUBR), :].astype(jnp.bfloat16)
            )
        local_copy = pltpu.make_async_copy(
            buf, out_ref.at[pl.ds(my_x * M_PER, M_PER), :], local_sem
        )
        local_copy.start()

        recvd_rows = other_x * M_PER + my_y * HALF
        rdma_b = []
        rcopies = []
        for i in range(NC):
            rdma_a[i].wait_recv()
            rows = recvd_rows + OFFS[i]
            r = pltpu.make_async_remote_copy(
                src_ref=rbuf.at[pl.ds(OFFS[i], CHUNKS[i]), :],
                dst_ref=out_ref.at[pl.ds(rows, CHUNKS[i]), :],
                send_sem=send_b.at[i],
                recv_sem=recv_b.at[i],
                device_id=(my_x, other_y),
                device_id_type=pl.DeviceIdType.MESH,
            )
            r.start()
            rdma_b.append(r)
            c = pltpu.make_async_copy(
                rbuf.at[pl.ds(OFFS[i], CHUNKS[i]), :],
                out_ref.at[pl.ds(rows, CHUNKS[i]), :],
                rcopy_sems.at[i],
            )
            c.start()
            rcopies.append(c)

        for i in range(NC):
            rdma_b[i].wait_recv()
        for i in range(NC):
            rdma_a[i].wait_send()
            rdma_b[i].wait_send()
            rcopies[i].wait()
        local_copy.wait()

    return pl.pallas_call(
        body,
        out_shape=jax.ShapeDtypeStruct((2 * M_PER, N), jnp.bfloat16),
        in_specs=[pl.BlockSpec(memory_space=pl.ANY)],
        out_specs=pl.BlockSpec(memory_space=pl.ANY),
        scratch_shapes=[
            pltpu.VMEM((M_PER, N), jnp.float32),
            pltpu.VMEM((M_PER, N), jnp.bfloat16),
            pltpu.VMEM((HALF, N), jnp.bfloat16),
            pltpu.SemaphoreType.DMA((NC,)),
            pltpu.SemaphoreType.DMA((NC,)),
            pltpu.SemaphoreType.DMA((NC,)),
            pltpu.SemaphoreType.DMA((NC,)),
            pltpu.SemaphoreType.DMA((2 * SUB,)),
            pltpu.SemaphoreType.DMA,
            pltpu.SemaphoreType.DMA((NC,)),
        ],
        compiler_params=pltpu.CompilerParams(collective_id=0),
    )(x)


# device time: 61578 ns/iter; 1.0071x vs baseline; 1.0071x over previous
import jax
import jax.numpy as jnp
from jax import lax
from jax.experimental import pallas as pl
from jax.experimental.pallas import tpu as pltpu

M_PER = 4096
N = 1024
HALF = M_PER // 2

CHUNKS = [128] * 16
OFFS = [sum(CHUNKS[:i]) for i in range(len(CHUNKS))]
NC = len(CHUNKS)
assert OFFS[-1] + CHUNKS[-1] == HALF

SUB = 4
SUBR = HALF // SUB


def kernel(x):
    def body(x_ref, out_ref, xvm, buf, send_a, recv_a, send_b, recv_b,
             in_sems, local_sem):
        my_x = lax.axis_index("x")
        my_y = lax.axis_index("y")
        other_x = 1 - my_x
        other_y = 1 - my_y

        in_dmas = []
        for j in range(2 * SUB):
            half = my_y * HALF if j < SUB else other_y * HALF
            r0 = half + (j % SUB) * SUBR
            c = pltpu.make_async_copy(
                x_ref.at[pl.ds(r0, SUBR), :], xvm.at[pl.ds(r0, SUBR), :],
                in_sems.at[j],
            )
            c.start()
            in_dmas.append(c)

        barrier_sem = pltpu.get_barrier_semaphore()
        pl.semaphore_signal(barrier_sem, inc=1, device_id=(other_x, my_y),
                            device_id_type=pl.DeviceIdType.MESH)
        pl.semaphore_signal(barrier_sem, inc=1, device_id=(my_x, other_y),
                            device_id_type=pl.DeviceIdType.MESH)
        pl.semaphore_wait(barrier_sem, 2)

        my_half_rows = my_x * M_PER + my_y * HALF
        rdma_a = []
        subs_waited = 0
        for i in range(NC):
            need = (OFFS[i] + CHUNKS[i] + SUBR - 1) // SUBR
            while subs_waited < need:
                in_dmas[subs_waited].wait()
                subs_waited += 1
            r0 = my_y * HALF + OFFS[i]
            buf[pl.ds(r0, CHUNKS[i]), :] = (
                xvm[pl.ds(r0, CHUNKS[i]), :].astype(jnp.bfloat16)
            )
            r = pltpu.make_async_remote_copy(
                src_ref=buf.at[pl.ds(r0, CHUNKS[i]), :],
                dst_ref=out_ref.at[pl.ds(my_half_rows + OFFS[i], CHUNKS[i]), :],
                send_sem=send_a.at[i],
                recv_sem=recv_a.at[i],
                device_id=(other_x, my_y),
                device_id_type=pl.DeviceIdType.MESH,
            )
            r.start()
            rdma_a.append(r)

        for j in range(SUB):
            in_dmas[SUB + j].wait()
            r0 = other_y * HALF + j * SUBR
            buf[pl.ds(r0, SUBR), :] = (
                xvm[pl.ds(r0, SUBR), :].astype(jnp.bfloat16)
            )
        local_copy = pltpu.make_async_copy(
            buf, out_ref.at[pl.ds(my_x * M_PER, M_PER), :], local_sem
        )
        local_copy.start()

        recvd_rows = other_x * M_PER + my_y * HALF
        rdma_b = []
        for i in range(NC):
            rdma_a[i].wait_recv()
            rows = recvd_rows + OFFS[i]
            r = pltpu.make_async_remote_copy(
                src_ref=out_ref.at[pl.ds(rows, CHUNKS[i]), :],
                dst_ref=out_ref.at[pl.ds(rows, CHUNKS[i]), :],
                send_sem=send_b.at[i],
                recv_sem=recv_b.at[i],
                device_id=(my_x, other_y),
                device_id_type=pl.DeviceIdType.MESH,
            )
            r.start()
            rdma_b.append(r)

        for i in range(NC):
            rdma_b[i].wait_recv()
        for i in range(NC):
            rdma_a[i].wait_send()
            rdma_b[i].wait_send()
        local_copy.wait()

    return pl.pallas_call(
        body,
        out_shape=jax.ShapeDtypeStruct((2 * M_PER, N), jnp.bfloat16),
        in_specs=[pl.BlockSpec(memory_space=pl.ANY)],
        out_specs=pl.BlockSpec(memory_space=pl.ANY),
        scratch_shapes=[
            pltpu.VMEM((M_PER, N), jnp.float32),
            pltpu.VMEM((M_PER, N), jnp.bfloat16),
            pltpu.SemaphoreType.DMA((NC,)),
            pltpu.SemaphoreType.DMA((NC,)),
            pltpu.SemaphoreType.DMA((NC,)),
            pltpu.SemaphoreType.DMA((NC,)),
            pltpu.SemaphoreType.DMA((2 * SUB,)),
            pltpu.SemaphoreType.DMA,
        ],
        compiler_params=pltpu.CompilerParams(collective_id=0),
    )(x)
